# baseline (speedup 1.0000x reference)
import functools

import jax
import jax.numpy as jnp
from jax import lax
from jax.experimental import pallas as pl
from jax.experimental.pallas import tpu as pltpu
from jax.experimental.pallas import tpu_sc as plsc

D = 1024
LANES = 16
NC, NS = 2, 16
NW = NC * NS
CHUNK = 16
NPE = 2
NO = 4


def _sc_gather_add(x2d, idx, table):
    rows = x2d.shape[0]
    b_per_w = rows // NW
    n_chunks = b_per_w // CHUNK
    mesh = plsc.VectorSubcoreMesh(core_axis_name="c", subcore_axis_name="s")

    buf = lambda: pltpu.VMEM((CHUNK, D), jnp.float32)
    sem = pltpu.SemaphoreType.DMA
    @functools.partial(
        pl.kernel,
        mesh=mesh,
        out_type=jax.ShapeDtypeStruct((rows, D), jnp.float32),
        scratch_types=[pltpu.VMEM((b_per_w,), jnp.int32)]
        + [buf() for _ in range(NPE + NO)]
        + [sem] * (NPE + 2 * NO),
    )
    def k(table_hbm, idx_hbm, x_hbm, out_hbm, idx_v, *bufs_and_sems):
        pe_v = bufs_and_sems[:NPE]
        o_v = bufs_and_sems[NPE:NPE + NO]
        gsem = bufs_and_sems[NPE + NO:2 * NPE + NO]
        xsem = bufs_and_sems[2 * NPE + NO:2 * NPE + 2 * NO]
        osem = bufs_and_sems[2 * NPE + 2 * NO:]

        wid = lax.axis_index("s") * NC + lax.axis_index("c")
        base = wid * b_per_w
        pltpu.sync_copy(idx_hbm.at[pl.ds(base, b_per_w)], idx_v)

        def gather(c, b):
            return pltpu.make_async_copy(
                table_hbm.at[idx_v.at[pl.ds(c * CHUNK, CHUNK)]], pe_v[b], gsem[b]
            )

        def x_copy(c, s):
            return pltpu.make_async_copy(
                x_hbm.at[pl.ds(base + c * CHUNK, CHUNK)], o_v[s], xsem[s]
            )

        def out_copy(c, s):
            return pltpu.make_async_copy(
                o_v[s], out_hbm.at[pl.ds(base + c * CHUNK, CHUNK)], osem[s]
            )

        for c in range(NPE):
            gather(c, c).start()
        for c in range(NPE):
            x_copy(c, c).start()

        @pl.loop(0, n_chunks, step=NO)
        def _quad(c0):
            for u in range(NO):
                c = c0 + u
                b = u % NPE
                s = u
                gather(c, b).wait()
                x_copy(c, s).wait()

                @pl.loop(0, CHUNK)
                def _row(r):
                    for j in range(D // LANES):
                        sl = (r, pl.ds(j * LANES, LANES))
                        plsc.addupdate(o_v[s].at[sl], pe_v[b][sl])

                out_copy(c, s).start()

                @pl.when(c + NPE < n_chunks)
                def _():
                    gather(c + NPE, b).start()

                @pl.when(c >= NO - NPE)
                def _():
                    out_copy(c - (NO - NPE), (s + NPE) % NO).wait()

                @pl.when(c + NPE < n_chunks)
                def _():
                    x_copy(c + NPE, (s + NPE) % NO).start()

        for u in range(NO - NPE, NO):
            out_copy(n_chunks - NO + u, u).wait()

    return k(table, idx, x2d)


def kernel(x, position_ids, pe_table):
    b, lp1, d = x.shape
    pos = jnp.concatenate(
        [jnp.zeros((b, 1), dtype=jnp.int32), position_ids.astype(jnp.int32)],
        axis=1,
    ).reshape(-1)
    x2d = x.reshape(b * lp1, d)
    out = _sc_gather_add(x2d, pos, pe_table)
    return out.reshape(b, lp1, d)
